# Initial kernel scaffold; baseline (speedup 1.0000x reference)
import jax
import jax.numpy as jnp
from jax.experimental import pallas as pl


def kernel(x, lengths, index):
    raise NotImplementedError("write your pallas kernel here")


# speedup vs baseline: 180.9304x; 180.9304x over previous
import functools

import jax
import jax.numpy as jnp
from jax import lax
from jax.experimental import pallas as pl
from jax.experimental.pallas import tpu as pltpu
from jax.experimental.pallas import tpu_sc as plsc

L = 16
NW = 32
TOTAL = 1_000_000
MAIN = (TOTAL // NW) // L * L
TAIL = TOTAL - NW * MAIN
CHUNK = MAIN + TAIL
NTYPES = 100
LPAD = 128

_mesh = plsc.VectorSubcoreMesh(core_axis_name="c", subcore_axis_name="s")


@functools.partial(
    pl.kernel,
    out_type=jax.ShapeDtypeStruct((TOTAL,), jnp.float32),
    mesh=_mesh,
    scratch_types=[
        pltpu.VMEM((CHUNK,), jnp.float32),
        pltpu.VMEM((CHUNK,), jnp.int32),
        pltpu.VMEM((LPAD,), jnp.float32),
        pltpu.SemaphoreType.DMA,
    ],
    compiler_params=pltpu.CompilerParams(needs_layout_passes=False),
)
def _inforate_sc(x_hbm, len_hbm, idx_hbm, out_hbm, xv, iv, rv, sem):
    wid = lax.axis_index("s") * 2 + lax.axis_index("c")
    base = wid * MAIN

    cx = pltpu.async_copy(x_hbm.at[pl.ds(base, MAIN)], xv.at[pl.ds(0, MAIN)], sem)
    ci = pltpu.async_copy(idx_hbm.at[pl.ds(base, MAIN)], iv.at[pl.ds(0, MAIN)], sem)
    cl = pltpu.async_copy(len_hbm, rv, sem)
    cl.wait()
    cx.wait()
    ci.wait()

    for k in range(LPAD // L):
        s = pl.ds(k * L, L)
        rv[s] = 1.0 / rv[s]

    last = wid == NW - 1

    @pl.when(last)
    def _tail_in():
        pltpu.sync_copy(x_hbm.at[pl.ds(NW * MAIN, TAIL)], xv.at[pl.ds(MAIN, TAIL)])
        pltpu.sync_copy(idx_hbm.at[pl.ds(NW * MAIN, TAIL)], iv.at[pl.ds(MAIN, TAIL)])

    def body(j, carry):
        s = pl.ds(j * L, L)
        r = plsc.load_gather(rv, [iv[s]])
        xv[s] = xv[s] * r
        return carry

    lax.fori_loop(0, MAIN // L, body, 0, unroll=4)

    @pl.when(last)
    def _tail_compute():
        for t in range(TAIL // L):
            s = pl.ds(MAIN + t * L, L)
            r = plsc.load_gather(rv, [iv[s]])
            xv[s] = xv[s] * r
        pltpu.sync_copy(xv.at[pl.ds(MAIN, TAIL)], out_hbm.at[pl.ds(NW * MAIN, TAIL)])

    pltpu.sync_copy(xv.at[pl.ds(0, MAIN)], out_hbm.at[pl.ds(base, MAIN)])


def kernel(x, lengths, index):
    lengths_padded = jnp.pad(lengths, (0, LPAD - NTYPES), constant_values=1.0)
    return _inforate_sc(x, lengths_padded, index)


# speedup vs baseline: 276.8865x; 1.5303x over previous
import functools

import jax
import jax.numpy as jnp
from jax import lax
from jax.experimental import pallas as pl
from jax.experimental.pallas import tpu as pltpu
from jax.experimental.pallas import tpu_sc as plsc

L = 16
NW = 32
TOTAL = 1_000_000
MAIN = (TOTAL // NW) // L * L
TAIL = TOTAL - NW * MAIN
CHUNK = MAIN + TAIL
NTYPES = 100
LPAD = 128

_mesh = plsc.VectorSubcoreMesh(core_axis_name="c", subcore_axis_name="s")


@functools.partial(
    pl.kernel,
    out_type=jax.ShapeDtypeStruct((TOTAL,), jnp.float32),
    mesh=_mesh,
    scratch_types=[
        pltpu.VMEM((CHUNK,), jnp.float32),
        pltpu.VMEM((CHUNK,), jnp.int32),
        pltpu.VMEM((CHUNK,), jnp.float32),
        pltpu.VMEM((LPAD,), jnp.float32),
        pltpu.SemaphoreType.DMA,
    ],
    compiler_params=pltpu.CompilerParams(needs_layout_passes=False),
)
def _inforate_sc(x_hbm, len_hbm, idx_hbm, out_hbm, xv, iv, ov, rv, sem):
    wid = lax.axis_index("s") * 2 + lax.axis_index("c")
    base = wid * MAIN

    cx = pltpu.async_copy(x_hbm.at[pl.ds(base, MAIN)], xv.at[pl.ds(0, MAIN)], sem)
    ci = pltpu.async_copy(idx_hbm.at[pl.ds(base, MAIN)], iv.at[pl.ds(0, MAIN)], sem)
    cl = pltpu.async_copy(len_hbm, rv, sem)
    cl.wait()
    cx.wait()
    ci.wait()

    for k in range(LPAD // L):
        s = pl.ds(k * L, L)
        rv[s] = 1.0 / rv[s]

    last = wid == NW - 1

    @pl.when(last)
    def _tail_in():
        pltpu.sync_copy(x_hbm.at[pl.ds(NW * MAIN, TAIL)], xv.at[pl.ds(MAIN, TAIL)])
        pltpu.sync_copy(idx_hbm.at[pl.ds(NW * MAIN, TAIL)], iv.at[pl.ds(MAIN, TAIL)])

    @plsc.parallel_loop(0, MAIN, L, unroll=8)
    def _main(i):
        s = pl.ds(i, L)
        r = plsc.load_gather(rv, [iv[s]])
        ov[s] = xv[s] * r

    @pl.when(last)
    def _tail_compute():
        for t in range(TAIL // L):
            s = pl.ds(MAIN + t * L, L)
            r = plsc.load_gather(rv, [iv[s]])
            ov[s] = xv[s] * r
        pltpu.sync_copy(ov.at[pl.ds(MAIN, TAIL)], out_hbm.at[pl.ds(NW * MAIN, TAIL)])

    pltpu.sync_copy(ov.at[pl.ds(0, MAIN)], out_hbm.at[pl.ds(base, MAIN)])


def kernel(x, lengths, index):
    lengths_padded = jnp.pad(lengths, (0, LPAD - NTYPES), constant_values=1.0)
    return _inforate_sc(x, lengths_padded, index)


# speedup vs baseline: 280.6945x; 1.0138x over previous
import functools

import jax
import jax.numpy as jnp
from jax import lax
from jax.experimental import pallas as pl
from jax.experimental.pallas import tpu as pltpu
from jax.experimental.pallas import tpu_sc as plsc

L = 16
NW = 32
TOTAL = 1_000_000
NB = 4
S = 7808
MAIN = NB * S
TAIL = TOTAL - NW * MAIN
NTYPES = 100
LPAD = 128

_mesh = plsc.VectorSubcoreMesh(core_axis_name="c", subcore_axis_name="s")


@functools.partial(
    pl.kernel,
    out_type=jax.ShapeDtypeStruct((TOTAL,), jnp.float32),
    mesh=_mesh,
    scratch_types=[
        pltpu.VMEM((S,), jnp.float32),
        pltpu.VMEM((S,), jnp.float32),
        pltpu.VMEM((S,), jnp.int32),
        pltpu.VMEM((S,), jnp.int32),
        pltpu.VMEM((S,), jnp.float32),
        pltpu.VMEM((S,), jnp.float32),
        pltpu.VMEM((LPAD,), jnp.float32),
        pltpu.SemaphoreType.DMA,
        pltpu.SemaphoreType.DMA,
        pltpu.SemaphoreType.DMA,
        pltpu.SemaphoreType.DMA,
        pltpu.SemaphoreType.DMA,
    ],
    compiler_params=pltpu.CompilerParams(needs_layout_passes=False),
)
def _inforate_sc(x_hbm, len_hbm, idx_hbm, out_hbm,
                 xv0, xv1, iv0, iv1, ov0, ov1, rv,
                 si0, si1, sl, so0, so1):
    wid = lax.axis_index("s") * 2 + lax.axis_index("c")
    base = wid * MAIN
    xs, ivs, ovs = [xv0, xv1], [iv0, iv1], [ov0, ov1]
    sin, son = [si0, si1], [so0, so1]

    def issue_in(b):
        off = base + b * S
        cx = pltpu.async_copy(x_hbm.at[pl.ds(off, S)], xs[b % 2], sin[b % 2])
        ci = pltpu.async_copy(idx_hbm.at[pl.ds(off, S)], ivs[b % 2], sin[b % 2])
        return cx, ci

    ins = {0: issue_in(0)}
    cl = pltpu.async_copy(len_hbm, rv, sl)
    cl.wait()
    for k in range(LPAD // L):
        s = pl.ds(k * L, L)
        rv[s] = 1.0 / rv[s]

    outs = {}
    for b in range(NB):
        if b + 1 < NB:
            ins[b + 1] = issue_in(b + 1)
        cx, ci = ins.pop(b)
        cx.wait()
        ci.wait()
        if b >= 2:
            outs.pop(b - 2).wait()
        xv, iv, ov = xs[b % 2], ivs[b % 2], ovs[b % 2]

        @plsc.parallel_loop(0, S, L, unroll=8)
        def _blk(i):
            s = pl.ds(i, L)
            r = plsc.load_gather(rv, [iv[s]])
            ov[s] = xv[s] * r

        outs[b] = pltpu.async_copy(ov, out_hbm.at[pl.ds(base + b * S, S)],
                                   son[b % 2])

    for b in sorted(outs):
        outs.pop(b).wait()

    @pl.when(wid == NW - 1)
    def _tail():
        toff = NW * MAIN
        pltpu.sync_copy(x_hbm.at[pl.ds(toff, TAIL)], xv0.at[pl.ds(0, TAIL)])
        pltpu.sync_copy(idx_hbm.at[pl.ds(toff, TAIL)], iv0.at[pl.ds(0, TAIL)])
        for t in range(TAIL // L):
            s = pl.ds(t * L, L)
            r = plsc.load_gather(rv, [iv0[s]])
            ov0[s] = xv0[s] * r
        pltpu.sync_copy(ov0.at[pl.ds(0, TAIL)], out_hbm.at[pl.ds(toff, TAIL)])


def kernel(x, lengths, index):
    lengths_padded = jnp.pad(lengths, (0, LPAD - NTYPES), constant_values=1.0)
    return _inforate_sc(x, lengths_padded, index)
